# baseline (speedup 1.0000x reference)
import functools
import math

import jax
import jax.numpy as jnp
from jax import lax
from jax.experimental import pallas as pl
from jax.experimental.pallas import tpu as pltpu
from jax.experimental.pallas import tpu_sc as plsc

B = 4096
S = 200
EMB = 32
SCALE = math.sqrt(float(EMB))
NW = 32
BPW = B // NW
TOK_PW = BPW * S
SLAB = EMB * B
TILE = 1024
CH_STRIDE = 32 * TILE


def _emb_body(tok_hbm, table_hbm, out_hbm, tok_v, idx0, idx1, rows0, rows1,
              tbuf0, tbuf1, gsem0, gsem1, wsem0, wsem1):
    wid = lax.axis_index("s") * 2 + lax.axis_index("c")
    tok_base = wid * TOK_PW

    pltpu.sync_copy(tok_hbm.at[pl.ds(tok_base, TOK_PW)], tok_v)

    ci = lax.iota(jnp.int32, 16)
    jb = ci * S

    def build_idx(s, idx_v):
        for m in range(8):
            t16 = plsc.load_gather(tok_v, [jb + (m * 16 * S + s)])
            idx_v[pl.ds(m * 16, 16)] = t16

    def transpose_rows(rows_v, tbuf):
        def m_body(m, _):
            rib = ci + 16 * m
            for c in range(EMB):
                v = plsc.load_gather(rows_v, [rib, ci * 0 + c]) * SCALE
                tbuf[pl.ds((c // 8) * TILE + (c % 8) * 128 + 16 * m, 16)] = v
            return 0
        lax.fori_loop(0, 8, m_body, 0)

    def issue_writes(s, tbuf, wsem):
        base = s * SLAB + wid * TILE
        for ch in range(4):
            pltpu.async_copy(tbuf.at[pl.ds(ch * TILE, TILE)],
                             out_hbm.at[pl.ds(base + ch * CH_STRIDE, TILE)],
                             wsem)

    def drain_writes(wsem, tbuf):
        pltpu.make_async_copy(out_hbm.at[pl.ds(0, 4 * TILE)], tbuf, wsem).wait()

    build_idx(0, idx0)
    g0 = pltpu.async_copy(table_hbm.at[idx0], rows0, gsem0)

    def pair_body(t, _):
        s0 = 2 * t
        s1 = s0 + 1
        build_idx(s1, idx1)
        pltpu.async_copy(table_hbm.at[idx1], rows1, gsem1)
        pltpu.make_async_copy(table_hbm.at[idx0], rows0, gsem0).wait()

        transpose_rows(rows0, tbuf0)
        issue_writes(s0, tbuf0, wsem0)

        build_idx(jnp.minimum(s0 + 2, S - 2), idx0)
        pltpu.async_copy(table_hbm.at[idx0], rows0, gsem0)
        pltpu.make_async_copy(table_hbm.at[idx1], rows1, gsem1).wait()

        transpose_rows(rows1, tbuf1)
        issue_writes(s1, tbuf1, wsem1)
        drain_writes(wsem0, tbuf0)
        drain_writes(wsem1, tbuf1)
        return 0

    lax.fori_loop(0, S // 2, pair_body, 0)
    pltpu.make_async_copy(table_hbm.at[idx0], rows0, gsem0).wait()


@jax.jit
def kernel(tokens, embedding_weight):
    tok_flat = tokens.reshape(B * S).astype(jnp.int32)
    mesh = plsc.VectorSubcoreMesh(core_axis_name="c", subcore_axis_name="s")
    run = functools.partial(
        pl.kernel,
        mesh=mesh,
        out_type=jax.ShapeDtypeStruct((B * S * EMB,), jnp.float32),
        scratch_types=[
            pltpu.VMEM((TOK_PW,), jnp.int32),
            pltpu.VMEM((BPW,), jnp.int32),
            pltpu.VMEM((BPW,), jnp.int32),
            pltpu.VMEM((BPW, EMB), jnp.float32),
            pltpu.VMEM((BPW, EMB), jnp.float32),
            pltpu.VMEM((4 * TILE,), jnp.float32),
            pltpu.VMEM((4 * TILE,), jnp.float32),
            pltpu.SemaphoreType.DMA,
            pltpu.SemaphoreType.DMA,
            pltpu.SemaphoreType.DMA,
            pltpu.SemaphoreType.DMA,
        ],
        compiler_params=pltpu.CompilerParams(
            use_tc_tiling_on_sc=False, needs_layout_passes=False),
    )(_emb_body)
    flat = run(tok_flat, embedding_weight)
    flat5 = flat.reshape(S, 4, B // 128, 8, 128)
    return flat5.transpose(2, 4, 0, 1, 3).reshape(B, S, EMB)


# speedup vs baseline: 1.3496x; 1.3496x over previous
import functools
import math

import jax
import jax.numpy as jnp
from jax import lax
from jax.experimental import pallas as pl
from jax.experimental.pallas import tpu as pltpu
from jax.experimental.pallas import tpu_sc as plsc

B = 4096
S = 200
EMB = 32
SCALE = math.sqrt(float(EMB))
NW = 32
BPW = B // NW
TOK_PW = BPW * S
SLAB = EMB * B
TILE = 1024
CH_STRIDE = 32 * TILE


def _emb_body(tok_hbm, table_hbm, out_hbm, tok_v, idx0, idx1, rows0, rows1,
              tbuf0, tbuf1, gsem0, gsem1, wsem0, wsem1):
    wid = lax.axis_index("s") * 2 + lax.axis_index("c")
    tok_base = wid * TOK_PW

    pltpu.sync_copy(tok_hbm.at[pl.ds(tok_base, TOK_PW)], tok_v)

    ci = lax.iota(jnp.int32, 16)
    jb = ci * S

    def build_idx(s, idx_v):
        for m in range(8):
            t16 = plsc.load_gather(tok_v, [jb + (m * 16 * S + s)])
            idx_v[pl.ds(m * 16, 16)] = t16

    def transpose_rows(rows_v, tbuf):
        def m_body(m, _):
            rib = ci + 16 * m
            vals = [plsc.load_gather(rows_v, [rib, ci * 0 + c]) * SCALE
                    for c in range(EMB)]
            for c in range(EMB):
                tbuf[pl.ds((c // 8) * TILE + (c % 8) * 128 + 16 * m, 16)] = vals[c]
            return 0
        lax.fori_loop(0, 8, m_body, 0)

    def issue_writes(s, tbuf, wsem):
        base = s * SLAB + wid * TILE
        for ch in range(4):
            pltpu.async_copy(tbuf.at[pl.ds(ch * TILE, TILE)],
                             out_hbm.at[pl.ds(base + ch * CH_STRIDE, TILE)],
                             wsem)

    def drain_writes(wsem, tbuf):
        pltpu.make_async_copy(out_hbm.at[pl.ds(0, 4 * TILE)], tbuf, wsem).wait()

    build_idx(0, idx0)
    g0 = pltpu.async_copy(table_hbm.at[idx0], rows0, gsem0)

    def pair_body(t, _):
        s0 = 2 * t
        s1 = s0 + 1
        build_idx(s1, idx1)
        pltpu.async_copy(table_hbm.at[idx1], rows1, gsem1)
        pltpu.make_async_copy(table_hbm.at[idx0], rows0, gsem0).wait()

        transpose_rows(rows0, tbuf0)
        issue_writes(s0, tbuf0, wsem0)

        build_idx(jnp.minimum(s0 + 2, S - 2), idx0)
        pltpu.async_copy(table_hbm.at[idx0], rows0, gsem0)
        pltpu.make_async_copy(table_hbm.at[idx1], rows1, gsem1).wait()

        transpose_rows(rows1, tbuf1)
        issue_writes(s1, tbuf1, wsem1)
        drain_writes(wsem0, tbuf0)
        drain_writes(wsem1, tbuf1)
        return 0

    lax.fori_loop(0, S // 2, pair_body, 0)
    pltpu.make_async_copy(table_hbm.at[idx0], rows0, gsem0).wait()


@jax.jit
def kernel(tokens, embedding_weight):
    tok_flat = tokens.reshape(B * S).astype(jnp.int32)
    mesh = plsc.VectorSubcoreMesh(core_axis_name="c", subcore_axis_name="s")
    run = functools.partial(
        pl.kernel,
        mesh=mesh,
        out_type=jax.ShapeDtypeStruct((B * S * EMB,), jnp.float32),
        scratch_types=[
            pltpu.VMEM((TOK_PW,), jnp.int32),
            pltpu.VMEM((BPW,), jnp.int32),
            pltpu.VMEM((BPW,), jnp.int32),
            pltpu.VMEM((BPW, EMB), jnp.float32),
            pltpu.VMEM((BPW, EMB), jnp.float32),
            pltpu.VMEM((4 * TILE,), jnp.float32),
            pltpu.VMEM((4 * TILE,), jnp.float32),
            pltpu.SemaphoreType.DMA,
            pltpu.SemaphoreType.DMA,
            pltpu.SemaphoreType.DMA,
            pltpu.SemaphoreType.DMA,
        ],
        compiler_params=pltpu.CompilerParams(
            use_tc_tiling_on_sc=False, needs_layout_passes=False),
    )(_emb_body)
    flat = run(tok_flat, embedding_weight)
    flat5 = flat.reshape(S, 4, B // 128, 8, 128)
    return flat5.transpose(2, 4, 0, 1, 3).reshape(B, S, EMB)
